# baseline (speedup 1.0000x reference)
import jax
import jax.numpy as jnp
from jax import lax
from jax.experimental import pallas as pl
from jax.experimental.pallas import tpu as pltpu
from jax.experimental.pallas import tpu_sc as plsc

_NC = 2
_NS = 16
_NW = _NC * _NS
_CK = 128
_GRP = 4
_DUMP = 128


def _sc_gather(table, idx3):
    nw, cpw, ck = idx3.shape
    d = table.shape[1]
    mesh = plsc.VectorSubcoreMesh(core_axis_name="c", subcore_axis_name="s")

    def body(table_hbm, idx_hbm, out_hbm, idx_v, rows_v, sem):
        wid = lax.axis_index("s") * _NC + lax.axis_index("c")
        pltpu.sync_copy(idx_hbm.at[wid], idx_v)
        base = wid * (cpw * ck)

        def group(gi, carry):
            g = gi * _GRP
            descs = [
                pltpu.async_copy(
                    table_hbm.at[idx_v.at[g + b]],
                    rows_v.at[pl.ds(b * ck, ck)],
                    sem,
                )
                for b in range(_GRP)
            ]
            for de in descs:
                de.wait()
            pltpu.sync_copy(rows_v, out_hbm.at[pl.ds(base + g * ck, _GRP * ck)])
            return carry

        lax.fori_loop(0, cpw // _GRP, group, 0)

    return pl.kernel(
        body,
        out_type=jax.ShapeDtypeStruct((nw * cpw * ck, d), jnp.float32),
        mesh=mesh,
        scratch_types=[
            pltpu.VMEM((cpw, ck), jnp.int32),
            pltpu.VMEM((_GRP * ck, d), jnp.float32),
            pltpu.SemaphoreType.DMA,
        ],
    )(table, idx3)


def _sc_scatter_add(vals, idx4, zeros_init):
    nc, ns, cpw, ck = idx4.shape
    rl, d = zeros_init.shape
    rsub = rl // _NS
    mesh = plsc.VectorSubcoreMesh(core_axis_name="c", subcore_axis_name="s")

    def body(vals_hbm, idx_hbm, zero_hbm, out_hbm, idx_v, rows_v, acc_sh):
        cid = lax.axis_index("c")
        sid = lax.axis_index("s")
        pltpu.sync_copy(
            zero_hbm.at[pl.ds(sid * rsub, rsub)],
            acc_sh.at[pl.ds(sid * rsub, rsub)],
        )
        plsc.subcore_barrier()
        pltpu.sync_copy(idx_hbm.at[cid, sid], idx_v)
        base = sid * (cpw * ck)

        def group(gi, carry):
            g = gi * _GRP
            pltpu.sync_copy(vals_hbm.at[pl.ds(base + g * ck, _GRP * ck)], rows_v)
            for b in range(_GRP):
                pltpu.sync_copy(
                    rows_v.at[pl.ds(b * ck, ck)],
                    acc_sh.at[idx_v.at[g + b]],
                    add=True,
                )
            return carry

        lax.fori_loop(0, cpw // _GRP, group, 0)
        plsc.subcore_barrier()
        pltpu.sync_copy(
            acc_sh.at[pl.ds(sid * rsub, rsub)],
            out_hbm.at[cid, pl.ds(sid * rsub, rsub)],
        )

    return pl.kernel(
        body,
        out_type=jax.ShapeDtypeStruct((nc, rl, d), jnp.float32),
        mesh=mesh,
        scratch_types=[
            pltpu.VMEM((cpw, ck), jnp.int32),
            pltpu.VMEM((_GRP * ck, d), jnp.float32),
            pltpu.VMEM_SHARED((rl, d), jnp.float32),
        ],
    )(vals, idx4, zeros_init)


def _ln(x, g, b):
    m = jnp.mean(x, axis=-1, keepdims=True)
    v = jnp.mean((x - m) ** 2, axis=-1, keepdims=True)
    return (x - m) * lax.rsqrt(v + 1e-5) * g + b


def _dot(a, b):
    return jnp.dot(a, b, preferred_element_type=jnp.float32,
                   precision=lax.Precision.HIGHEST)


def _full(shape):
    return pl.BlockSpec(shape, lambda i: (0,) * len(shape))


def _mlp_tail(h1, w2, b2, g2, be2, w3, b3):
    h2 = jnp.maximum(_ln(_dot(h1, w2) + b2, g2, be2), 0.0)
    return _dot(h2, w3) + b3, h2


def _pad64(x, blk):
    return jnp.concatenate([x, jnp.zeros((blk, 64), jnp.float32)], axis=1)


def _tc_input_mlp(x, p, blk):
    r = x.shape[0]
    (w1, b1, g1, be1), (w2, b2, g2, be2) = p["hid"]
    w3, b3 = p["out"]

    def body(x_ref, w1_r, b1_r, g1_r, be1_r, w2_r, b2_r, g2_r, be2_r,
             w3_r, b3_r, out_ref):
        h1 = jnp.maximum(_ln(_dot(x_ref[...], w1_r[...]) + b1_r[...],
                             g1_r[...], be1_r[...]), 0.0)
        out, _ = _mlp_tail(h1, w2_r[...], b2_r[...], g2_r[...], be2_r[...],
                           w3_r[...], b3_r[...])
        out_ref[...] = _pad64(out, blk)

    return pl.pallas_call(
        body,
        grid=(r // blk,),
        in_specs=[
            pl.BlockSpec((blk, 128), lambda i: (i, 0)),
            _full(w1.shape), _full(b1.shape), _full(g1.shape), _full(be1.shape),
            _full(w2.shape), _full(b2.shape), _full(g2.shape), _full(be2.shape),
            _full(w3.shape), _full(b3.shape),
        ],
        out_specs=pl.BlockSpec((blk, 128), lambda i: (i, 0)),
        out_shape=jax.ShapeDtypeStruct((r, 128), jnp.float32),
    )(x, w1, b1, g1, be1, w2, b2, g2, be2, w3, b3)


def _tc_msg_mlp(sg, ce, p, blk):
    e = sg.shape[0]
    w1s, w1c, b1, g1, be1 = p["l1"]
    (w2, b2, g2, be2) = p["l2"]
    w3, b3 = p["out"]

    def body(sg_ref, ce_ref, w1s_r, w1c_r, b1_r, g1_r, be1_r,
             w2_r, b2_r, g2_r, be2_r, w3_r, b3_r, out_ref):
        pre1 = _dot(sg_ref[...], w1s_r[...]) + _dot(ce_ref[...], w1c_r[...]) + b1_r[...]
        h1 = jnp.maximum(_ln(pre1, g1_r[...], be1_r[...]), 0.0)
        out, _ = _mlp_tail(h1, w2_r[...], b2_r[...], g2_r[...], be2_r[...],
                           w3_r[...], b3_r[...])
        out_ref[...] = jnp.concatenate(
            [jnp.tanh(out),
             jnp.ones((blk, 1), jnp.float32),
             jnp.zeros((blk, 63), jnp.float32)], axis=1)

    return pl.pallas_call(
        body,
        grid=(e // blk,),
        in_specs=[
            pl.BlockSpec((blk, 128), lambda i: (i, 0)),
            pl.BlockSpec((blk, 128), lambda i: (i, 0)),
            _full(w1s.shape), _full(w1c.shape), _full(b1.shape),
            _full(g1.shape), _full(be1.shape),
            _full(w2.shape), _full(b2.shape), _full(g2.shape), _full(be2.shape),
            _full(w3.shape), _full(b3.shape),
        ],
        out_specs=pl.BlockSpec((blk, 128), lambda i: (i, 0)),
        out_shape=jax.ShapeDtypeStruct((e, 128), jnp.float32),
    )(sg, ce, w1s, w1c, b1, g1, be1, w2, b2, g2, be2, w3, b3)


def _tc_update_mlp(msum, st, p, blk):
    r = st.shape[0]
    nbh = (r // 2) // blk
    w1m, w1s, b1, g1, be1 = p["l1"]
    (w2, b2, g2, be2) = p["l2"]
    w3, b3 = p["out"]

    def body(ms_ref, st_ref, w1m_r, w1s_r, b1_r, g1_r, be1_r,
             w2_r, b2_r, g2_r, be2_r, w3_r, b3_r, out_ref):
        m = ms_ref[0]
        inv = 1.0 / jnp.maximum(m[:, 64:65], 1.0)
        pre1 = (_dot(m * inv, w1m_r[...])
                + _dot(st_ref[...], w1s_r[...]) + b1_r[...])
        h1 = jnp.maximum(_ln(pre1, g1_r[...], be1_r[...]), 0.0)
        out, _ = _mlp_tail(h1, w2_r[...], b2_r[...], g2_r[...], be2_r[...],
                           w3_r[...], b3_r[...])
        out_ref[...] = _pad64(out, blk)

    return pl.pallas_call(
        body,
        grid=(r // blk,),
        in_specs=[
            pl.BlockSpec((1, blk, 128), lambda i: (i // nbh, i % nbh, 0)),
            pl.BlockSpec((blk, 128), lambda i: (i, 0)),
            _full(w1m.shape), _full(w1s.shape), _full(b1.shape),
            _full(g1.shape), _full(be1.shape),
            _full(w2.shape), _full(b2.shape), _full(g2.shape), _full(be2.shape),
            _full(w3.shape), _full(b3.shape),
        ],
        out_specs=pl.BlockSpec((blk, 128), lambda i: (i, 0)),
        out_shape=jax.ShapeDtypeStruct((r, 128), jnp.float32),
    )(msum, st, w1m, w1s, b1, g1, be1, w2, b2, g2, be2, w3, b3)


def _tc_output_mlp(st, p, blk, n_valid):
    r = st.shape[0]
    (w1, b1, g1, be1), (w2, b2, g2, be2) = p["hid"]
    w3, b3 = p["out"]
    wsig, bsig = p["sig"]

    def body(st_ref, w1_r, b1_r, g1_r, be1_r, w2_r, b2_r, g2_r, be2_r,
             w3_r, b3_r, ws_r, bs_r, act_ref, psum_ref):
        i = pl.program_id(0)
        h1 = jnp.maximum(_ln(_dot(st_ref[...], w1_r[...]) + b1_r[...],
                             g1_r[...], be1_r[...]), 0.0)
        out, h2 = _mlp_tail(h1, w2_r[...], b2_r[...], g2_r[...], be2_r[...],
                            w3_r[...], b3_r[...])
        act_ref[...] = jnp.tanh(out)
        sig = jax.nn.sigmoid(_dot(h2, ws_r[...]) + bs_r[...])
        rows = i * blk + lax.broadcasted_iota(jnp.int32, (blk, 1), 0)
        sig = jnp.where(rows < n_valid, sig, 0.0)
        psum_ref[...] = jnp.sum(sig).reshape(1, 1, 1)

    return pl.pallas_call(
        body,
        grid=(r // blk,),
        in_specs=[
            pl.BlockSpec((blk, 128), lambda i: (i, 0)),
            _full(w1.shape), _full(b1.shape), _full(g1.shape), _full(be1.shape),
            _full(w2.shape), _full(b2.shape), _full(g2.shape), _full(be2.shape),
            _full(w3.shape), _full(b3.shape),
            _full(wsig.shape), _full(bsig.shape),
        ],
        out_specs=[
            pl.BlockSpec((blk, 1), lambda i: (i, 0)),
            pl.BlockSpec((1, 1, 1), lambda i: (i, 0, 0)),
        ],
        out_shape=[
            jax.ShapeDtypeStruct((r, 1), jnp.float32),
            jax.ShapeDtypeStruct((r // blk, 1, 1), jnp.float32),
        ],
    )(st, w1, b1, g1, be1, w2, b2, g2, be2, w3, b3, wsig, bsig)


def _row2(v):
    return v.reshape(1, -1)


def _padrows(w, rows):
    return jnp.zeros((rows, w.shape[1]), jnp.float32).at[:w.shape[0]].set(w)


def kernel(state, node_features, edge_feature, edge_index, params_input,
           params_message, params_update, params_output):
    f32 = jnp.float32
    b, sd = state.shape
    nsv = sd // 2
    n = (nsv - 5) // 2
    e = edge_index.shape[1]

    blk_n = 1024
    n_pad = -(-(n + 1) // (2 * blk_n)) * (2 * blk_n)
    n_half = n_pad // 2
    rl = n_half + _DUMP
    epw = _GRP * _CK
    e_pad = -(-e // (_NW * epw)) * (_NW * epw)
    cpw = e_pad // (_NW * _CK)
    cpw2 = e_pad // (_NS * _CK)
    blk_e = 1024

    s0 = state[0]
    glob = jnp.concatenate([s0[0:5], s0[nsv:nsv + 5]])
    node_input = jnp.concatenate([
        node_features,
        jnp.broadcast_to(glob[None, :], (n, 10)),
        s0[5:5 + n][:, None],
        s0[5 + n:5 + 2 * n][:, None],
        s0[nsv + 5:nsv + 5 + n][:, None],
        s0[nsv + 5 + n:nsv + 5 + 2 * n][:, None],
    ], axis=1)
    ninp_tbl = jnp.zeros((n_pad, 128), f32).at[:n, :20].set(node_input)

    src = jnp.zeros((e_pad,), jnp.int32).at[:e].set(edge_index[0])
    dst = jnp.full((e_pad,), n, jnp.int32).at[:e].set(edge_index[1])
    src3 = src.reshape(_NW, cpw, _CK)
    dst2 = dst.reshape(_NS, cpw2, _CK)
    halves = []
    for c in range(_NC):
        loc = dst2 - c * n_half
        ok = (loc >= 0) & (loc < n_half)
        halves.append(jnp.where(ok, loc, n_half))
    dst4 = jnp.stack(halves)
    edat = jnp.zeros((e_pad,), f32).at[:e].set(edge_feature)

    def prep_plain(p, in_rows):
        (w1, b1, g1, be1), (w2, b2, g2, be2) = p["hidden"]
        return {
            "hid": [
                (_padrows(w1, in_rows), _row2(b1), _row2(g1), _row2(be1)),
                (w2, _row2(b2), _row2(g2), _row2(be2)),
            ],
            "out": (p["Wout"], _row2(p["bout"])),
        }

    pi = prep_plain(params_input, 128)

    wm1, bm1, gm1, bem1 = params_message["hidden"][0]
    pm = {
        "l1": (_padrows(wm1[:64], 128),
               _padrows(wm1[65:85], 128).at[20].set(wm1[64]),
               _row2(bm1), _row2(gm1), _row2(bem1)),
        "l2": tuple([params_message["hidden"][1][0]]
                    + [_row2(v) for v in params_message["hidden"][1][1:]]),
        "out": (params_message["Wout"], _row2(params_message["bout"])),
    }
    wu1, bu1, gu1, beu1 = params_update["hidden"][0]
    pu = {
        "l1": (_padrows(wu1[:64], 128), _padrows(wu1[64:], 128),
               _row2(bu1), _row2(gu1), _row2(beu1)),
        "l2": tuple([params_update["hidden"][1][0]]
                    + [_row2(v) for v in params_update["hidden"][1][1:]]),
        "out": (params_update["Wout"], _row2(params_update["bout"])),
    }
    po = prep_plain(params_output, 128)
    po["sig"] = (params_output["Wsig"], _row2(params_output["bsig"]))

    ce = _sc_gather(ninp_tbl, src3)
    ce = ce.at[:, 20].set(edat)

    node_state = _tc_input_mlp(ninp_tbl, pi, blk_n)

    zrl = jnp.zeros((rl, 128), f32)
    for _ in range(6):
        sg = _sc_gather(node_state, src3)
        msg = _tc_msg_mlp(sg, ce, pm, blk_e)
        msum = _sc_scatter_add(msg, dst4, zrl)
        node_state = _tc_update_mlp(msum, node_state, pu, blk_n)

    act, psum = _tc_output_mlp(node_state, po, blk_n, n)
    actions = act[:n, 0][None, :]
    sigmoids = (jnp.sum(psum) / n).reshape(1)
    return (actions, sigmoids)


# speedup vs baseline: 2.2624x; 2.2624x over previous
import jax
import jax.numpy as jnp
from jax import lax
from jax.experimental import pallas as pl
from jax.experimental.pallas import tpu as pltpu
from jax.experimental.pallas import tpu_sc as plsc

_NC = 2
_NS = 16
_NW = _NC * _NS
_CK = 128
_GRP = 4
_DUMP = 128


def _sc_gather(table, idx3):
    nw, cpw, ck = idx3.shape
    d = table.shape[1]
    mesh = plsc.VectorSubcoreMesh(core_axis_name="c", subcore_axis_name="s")

    def body(table_hbm, idx_hbm, out_hbm, idx_v, rows_v, sem):
        wid = lax.axis_index("s") * _NC + lax.axis_index("c")
        pltpu.sync_copy(idx_hbm.at[wid], idx_v)
        base = wid * (cpw * ck)

        def group(gi, carry):
            g = gi * _GRP
            descs = [
                pltpu.async_copy(
                    table_hbm.at[idx_v.at[g + b]],
                    rows_v.at[pl.ds(b * ck, ck)],
                    sem,
                )
                for b in range(_GRP)
            ]
            for de in descs:
                de.wait()
            pltpu.sync_copy(rows_v, out_hbm.at[pl.ds(base + g * ck, _GRP * ck)])
            return carry

        lax.fori_loop(0, cpw // _GRP, group, 0)

    return pl.kernel(
        body,
        out_type=jax.ShapeDtypeStruct((nw * cpw * ck, d), jnp.float32),
        mesh=mesh,
        scratch_types=[
            pltpu.VMEM((cpw, ck), jnp.int32),
            pltpu.VMEM((_GRP * ck, d), jnp.float32),
            pltpu.SemaphoreType.DMA,
        ],
    )(table, idx3)


def _sc_scatter_add(vals, idx4, zeros_init):
    nc, ns, cpw, ck = idx4.shape
    rl, d = zeros_init.shape
    rsub = rl // _NS
    mesh = plsc.VectorSubcoreMesh(core_axis_name="c", subcore_axis_name="s")

    def body(vals_hbm, idx_hbm, zero_hbm, out_hbm, idx_v, rows_v, acc_sh):
        cid = lax.axis_index("c")
        sid = lax.axis_index("s")
        pltpu.sync_copy(
            zero_hbm.at[pl.ds(sid * rsub, rsub)],
            acc_sh.at[pl.ds(sid * rsub, rsub)],
        )
        plsc.subcore_barrier()
        pltpu.sync_copy(idx_hbm.at[cid, sid], idx_v)
        base = sid * (cpw * ck)

        def group(gi, carry):
            g = gi * _GRP
            pltpu.sync_copy(vals_hbm.at[pl.ds(base + g * ck, _GRP * ck)], rows_v)
            for b in range(_GRP):
                pltpu.sync_copy(
                    rows_v.at[pl.ds(b * ck, ck)],
                    acc_sh.at[idx_v.at[g + b]],
                    add=True,
                )
            return carry

        lax.fori_loop(0, cpw // _GRP, group, 0)
        plsc.subcore_barrier()
        pltpu.sync_copy(
            acc_sh.at[pl.ds(sid * rsub, rsub)],
            out_hbm.at[cid, pl.ds(sid * rsub, rsub)],
        )

    return pl.kernel(
        body,
        out_type=jax.ShapeDtypeStruct((nc, rl, d), jnp.float32),
        mesh=mesh,
        scratch_types=[
            pltpu.VMEM((cpw, ck), jnp.int32),
            pltpu.VMEM((_GRP * ck, d), jnp.float32),
            pltpu.VMEM_SHARED((rl, d), jnp.float32),
        ],
    )(vals, idx4, zeros_init)


def _ln(x, g, b):
    m = jnp.mean(x, axis=-1, keepdims=True)
    v = jnp.mean((x - m) ** 2, axis=-1, keepdims=True)
    return (x - m) * lax.rsqrt(v + 1e-5) * g + b


def _dot(a, b):
    return jnp.dot(a, b, preferred_element_type=jnp.float32,
                   precision=lax.Precision.DEFAULT)


def _full(shape):
    return pl.BlockSpec(shape, lambda i: (0,) * len(shape))


def _mlp_tail(h1, w2, b2, g2, be2, w3, b3):
    h2 = jnp.maximum(_ln(_dot(h1, w2) + b2, g2, be2), 0.0)
    return _dot(h2, w3) + b3, h2


def _pad64(x, blk):
    return jnp.concatenate([x, jnp.zeros((blk, 64), jnp.float32)], axis=1)


def _tc_input_mlp(x, p, blk):
    r = x.shape[0]
    (w1, b1, g1, be1), (w2, b2, g2, be2) = p["hid"]
    w3, b3 = p["out"]

    def body(x_ref, w1_r, b1_r, g1_r, be1_r, w2_r, b2_r, g2_r, be2_r,
             w3_r, b3_r, out_ref):
        h1 = jnp.maximum(_ln(_dot(x_ref[...], w1_r[...]) + b1_r[...],
                             g1_r[...], be1_r[...]), 0.0)
        out, _ = _mlp_tail(h1, w2_r[...], b2_r[...], g2_r[...], be2_r[...],
                           w3_r[...], b3_r[...])
        out_ref[...] = _pad64(out, blk)

    return pl.pallas_call(
        body,
        grid=(r // blk,),
        in_specs=[
            pl.BlockSpec((blk, 128), lambda i: (i, 0)),
            _full(w1.shape), _full(b1.shape), _full(g1.shape), _full(be1.shape),
            _full(w2.shape), _full(b2.shape), _full(g2.shape), _full(be2.shape),
            _full(w3.shape), _full(b3.shape),
        ],
        out_specs=pl.BlockSpec((blk, 128), lambda i: (i, 0)),
        out_shape=jax.ShapeDtypeStruct((r, 128), jnp.float32),
    )(x, w1, b1, g1, be1, w2, b2, g2, be2, w3, b3)


def _tc_msg_mlp(sg, ce, p, blk):
    e = sg.shape[0]
    w1s, w1c, b1, g1, be1 = p["l1"]
    (w2, b2, g2, be2) = p["l2"]
    w3, b3 = p["out"]

    def body(sg_ref, ce_ref, w1s_r, w1c_r, b1_r, g1_r, be1_r,
             w2_r, b2_r, g2_r, be2_r, w3_r, b3_r, out_ref):
        pre1 = _dot(sg_ref[...], w1s_r[...]) + _dot(ce_ref[...], w1c_r[...]) + b1_r[...]
        h1 = jnp.maximum(_ln(pre1, g1_r[...], be1_r[...]), 0.0)
        out, _ = _mlp_tail(h1, w2_r[...], b2_r[...], g2_r[...], be2_r[...],
                           w3_r[...], b3_r[...])
        out_ref[...] = jnp.concatenate(
            [jnp.tanh(out),
             jnp.ones((blk, 1), jnp.float32),
             jnp.zeros((blk, 63), jnp.float32)], axis=1)

    return pl.pallas_call(
        body,
        grid=(e // blk,),
        in_specs=[
            pl.BlockSpec((blk, 128), lambda i: (i, 0)),
            pl.BlockSpec((blk, 128), lambda i: (i, 0)),
            _full(w1s.shape), _full(w1c.shape), _full(b1.shape),
            _full(g1.shape), _full(be1.shape),
            _full(w2.shape), _full(b2.shape), _full(g2.shape), _full(be2.shape),
            _full(w3.shape), _full(b3.shape),
        ],
        out_specs=pl.BlockSpec((blk, 128), lambda i: (i, 0)),
        out_shape=jax.ShapeDtypeStruct((e, 128), jnp.float32),
    )(sg, ce, w1s, w1c, b1, g1, be1, w2, b2, g2, be2, w3, b3)


def _tc_update_mlp(msum, st, p, blk):
    r = st.shape[0]
    nbh = (r // 2) // blk
    w1m, w1s, b1, g1, be1 = p["l1"]
    (w2, b2, g2, be2) = p["l2"]
    w3, b3 = p["out"]

    def body(ms_ref, st_ref, w1m_r, w1s_r, b1_r, g1_r, be1_r,
             w2_r, b2_r, g2_r, be2_r, w3_r, b3_r, out_ref):
        m = ms_ref[0]
        inv = 1.0 / jnp.maximum(m[:, 64:65], 1.0)
        pre1 = (_dot(m * inv, w1m_r[...])
                + _dot(st_ref[...], w1s_r[...]) + b1_r[...])
        h1 = jnp.maximum(_ln(pre1, g1_r[...], be1_r[...]), 0.0)
        out, _ = _mlp_tail(h1, w2_r[...], b2_r[...], g2_r[...], be2_r[...],
                           w3_r[...], b3_r[...])
        out_ref[...] = _pad64(out, blk)

    return pl.pallas_call(
        body,
        grid=(r // blk,),
        in_specs=[
            pl.BlockSpec((1, blk, 128), lambda i: (i // nbh, i % nbh, 0)),
            pl.BlockSpec((blk, 128), lambda i: (i, 0)),
            _full(w1m.shape), _full(w1s.shape), _full(b1.shape),
            _full(g1.shape), _full(be1.shape),
            _full(w2.shape), _full(b2.shape), _full(g2.shape), _full(be2.shape),
            _full(w3.shape), _full(b3.shape),
        ],
        out_specs=pl.BlockSpec((blk, 128), lambda i: (i, 0)),
        out_shape=jax.ShapeDtypeStruct((r, 128), jnp.float32),
    )(msum, st, w1m, w1s, b1, g1, be1, w2, b2, g2, be2, w3, b3)


def _tc_output_mlp(st, p, blk, n_valid):
    r = st.shape[0]
    (w1, b1, g1, be1), (w2, b2, g2, be2) = p["hid"]
    w3, b3 = p["out"]
    wsig, bsig = p["sig"]

    def body(st_ref, w1_r, b1_r, g1_r, be1_r, w2_r, b2_r, g2_r, be2_r,
             w3_r, b3_r, ws_r, bs_r, act_ref, psum_ref):
        i = pl.program_id(0)
        h1 = jnp.maximum(_ln(_dot(st_ref[...], w1_r[...]) + b1_r[...],
                             g1_r[...], be1_r[...]), 0.0)
        out, h2 = _mlp_tail(h1, w2_r[...], b2_r[...], g2_r[...], be2_r[...],
                            w3_r[...], b3_r[...])
        act_ref[...] = jnp.tanh(out)
        sig = jax.nn.sigmoid(_dot(h2, ws_r[...]) + bs_r[...])
        rows = i * blk + lax.broadcasted_iota(jnp.int32, (blk, 1), 0)
        sig = jnp.where(rows < n_valid, sig, 0.0)
        psum_ref[...] = jnp.sum(sig).reshape(1, 1, 1)

    return pl.pallas_call(
        body,
        grid=(r // blk,),
        in_specs=[
            pl.BlockSpec((blk, 128), lambda i: (i, 0)),
            _full(w1.shape), _full(b1.shape), _full(g1.shape), _full(be1.shape),
            _full(w2.shape), _full(b2.shape), _full(g2.shape), _full(be2.shape),
            _full(w3.shape), _full(b3.shape),
            _full(wsig.shape), _full(bsig.shape),
        ],
        out_specs=[
            pl.BlockSpec((blk, 1), lambda i: (i, 0)),
            pl.BlockSpec((1, 1, 1), lambda i: (i, 0, 0)),
        ],
        out_shape=[
            jax.ShapeDtypeStruct((r, 1), jnp.float32),
            jax.ShapeDtypeStruct((r // blk, 1, 1), jnp.float32),
        ],
    )(st, w1, b1, g1, be1, w2, b2, g2, be2, w3, b3, wsig, bsig)


def _row2(v):
    return v.reshape(1, -1)


def _padrows(w, rows):
    return jnp.zeros((rows, w.shape[1]), jnp.float32).at[:w.shape[0]].set(w)


def kernel(state, node_features, edge_feature, edge_index, params_input,
           params_message, params_update, params_output):
    f32 = jnp.float32
    b, sd = state.shape
    nsv = sd // 2
    n = (nsv - 5) // 2
    e = edge_index.shape[1]

    blk_n = 1024
    n_pad = -(-(n + 1) // (2 * blk_n)) * (2 * blk_n)
    n_half = n_pad // 2
    rl = n_half + _DUMP
    epw = _GRP * _CK
    e_pad = -(-e // (_NW * epw)) * (_NW * epw)
    cpw = e_pad // (_NW * _CK)
    cpw2 = e_pad // (_NS * _CK)
    blk_e = 1024

    s0 = state[0]
    glob = jnp.concatenate([s0[0:5], s0[nsv:nsv + 5]])
    node_input = jnp.concatenate([
        node_features,
        jnp.broadcast_to(glob[None, :], (n, 10)),
        s0[5:5 + n][:, None],
        s0[5 + n:5 + 2 * n][:, None],
        s0[nsv + 5:nsv + 5 + n][:, None],
        s0[nsv + 5 + n:nsv + 5 + 2 * n][:, None],
    ], axis=1)
    ninp_tbl = jnp.zeros((n_pad, 128), f32).at[:n, :20].set(node_input)

    src = jnp.zeros((e_pad,), jnp.int32).at[:e].set(edge_index[0])
    dst = jnp.full((e_pad,), n, jnp.int32).at[:e].set(edge_index[1])
    src3 = src.reshape(_NW, cpw, _CK)
    dst2 = dst.reshape(_NS, cpw2, _CK)
    halves = []
    for c in range(_NC):
        loc = dst2 - c * n_half
        ok = (loc >= 0) & (loc < n_half)
        halves.append(jnp.where(ok, loc, n_half))
    dst4 = jnp.stack(halves)
    edat = jnp.zeros((e_pad,), f32).at[:e].set(edge_feature)

    def prep_plain(p, in_rows):
        (w1, b1, g1, be1), (w2, b2, g2, be2) = p["hidden"]
        return {
            "hid": [
                (_padrows(w1, in_rows), _row2(b1), _row2(g1), _row2(be1)),
                (w2, _row2(b2), _row2(g2), _row2(be2)),
            ],
            "out": (p["Wout"], _row2(p["bout"])),
        }

    pi = prep_plain(params_input, 128)

    wm1, bm1, gm1, bem1 = params_message["hidden"][0]
    pm = {
        "l1": (_padrows(wm1[:64], 128),
               _padrows(wm1[65:85], 128).at[20].set(wm1[64]),
               _row2(bm1), _row2(gm1), _row2(bem1)),
        "l2": tuple([params_message["hidden"][1][0]]
                    + [_row2(v) for v in params_message["hidden"][1][1:]]),
        "out": (params_message["Wout"], _row2(params_message["bout"])),
    }
    wu1, bu1, gu1, beu1 = params_update["hidden"][0]
    pu = {
        "l1": (_padrows(wu1[:64], 128), _padrows(wu1[64:], 128),
               _row2(bu1), _row2(gu1), _row2(beu1)),
        "l2": tuple([params_update["hidden"][1][0]]
                    + [_row2(v) for v in params_update["hidden"][1][1:]]),
        "out": (params_update["Wout"], _row2(params_update["bout"])),
    }
    po = prep_plain(params_output, 128)
    po["sig"] = (params_output["Wsig"], _row2(params_output["bsig"]))

    ce = _sc_gather(ninp_tbl, src3)
    ce = ce.at[:, 20].set(edat)

    node_state = _tc_input_mlp(ninp_tbl, pi, blk_n)

    zrl = jnp.zeros((rl, 128), f32)
    for _ in range(6):
        sg = _sc_gather(node_state, src3)
        msg = _tc_msg_mlp(sg, ce, pm, blk_e)
        msum = _sc_scatter_add(msg, dst4, zrl)
        node_state = _tc_update_mlp(msum, node_state, pu, blk_n)

    act, psum = _tc_output_mlp(node_state, po, blk_n, n)
    actions = act[:n, 0][None, :]
    sigmoids = (jnp.sum(psum) / n).reshape(1)
    return (actions, sigmoids)


# speedup vs baseline: 2.3726x; 1.0487x over previous
import jax
import jax.numpy as jnp
from jax import lax
from jax.experimental import pallas as pl
from jax.experimental.pallas import tpu as pltpu
from jax.experimental.pallas import tpu_sc as plsc

_NC = 2
_NS = 16
_NW = _NC * _NS
_CK = 128
_GRP = 2
_DUMP = 128


def _sc_gather(table, idx3):
    nw, cpw, ck = idx3.shape
    d = table.shape[1]
    mesh = plsc.VectorSubcoreMesh(core_axis_name="c", subcore_axis_name="s")

    gsz = _GRP * ck

    def body(table_hbm, idx_hbm, out_hbm, idx_v, rows_a, rows_b, sem,
             sem_wa, sem_wb):
        wid = lax.axis_index("s") * _NC + lax.axis_index("c")
        pltpu.sync_copy(idx_hbm.at[wid], idx_v)
        base = wid * (cpw * ck)

        def run(g, rows_v, sem_w, first):
            @pl.when(jnp.logical_not(first))
            def _():
                pltpu.make_async_copy(
                    rows_v, out_hbm.at[pl.ds(base, gsz)], sem_w).wait()
            descs = [
                pltpu.async_copy(
                    table_hbm.at[idx_v.at[g + b]],
                    rows_v.at[pl.ds(b * ck, ck)],
                    sem,
                )
                for b in range(_GRP)
            ]
            for de in descs:
                de.wait()
            pltpu.async_copy(rows_v, out_hbm.at[pl.ds(base + g * ck, gsz)],
                             sem_w)

        def pair(pi, carry):
            g = pi * (2 * _GRP)
            run(g, rows_a, sem_wa, pi == 0)
            run(g + _GRP, rows_b, sem_wb, pi == 0)
            return carry

        lax.fori_loop(0, cpw // (2 * _GRP), pair, 0)
        pltpu.make_async_copy(rows_a, out_hbm.at[pl.ds(base, gsz)],
                              sem_wa).wait()
        pltpu.make_async_copy(rows_b, out_hbm.at[pl.ds(base, gsz)],
                              sem_wb).wait()

    return pl.kernel(
        body,
        out_type=jax.ShapeDtypeStruct((nw * cpw * ck, d), jnp.float32),
        mesh=mesh,
        scratch_types=[
            pltpu.VMEM((cpw, ck), jnp.int32),
            pltpu.VMEM((gsz, d), jnp.float32),
            pltpu.VMEM((gsz, d), jnp.float32),
            pltpu.SemaphoreType.DMA,
            pltpu.SemaphoreType.DMA,
            pltpu.SemaphoreType.DMA,
        ],
    )(table, idx3)


def _sc_scatter_add(vals, idx4, zeros_init):
    nc, ns, cpw, ck = idx4.shape
    rl, d = zeros_init.shape
    rsub = rl // _NS
    mesh = plsc.VectorSubcoreMesh(core_axis_name="c", subcore_axis_name="s")

    gsz = _GRP * ck
    npair = cpw // (2 * _GRP)

    def body(vals_hbm, idx_hbm, zero_hbm, out_hbm, idx_v, rows_a, rows_b,
             acc_sh, sem_a, sem_b):
        cid = lax.axis_index("c")
        sid = lax.axis_index("s")
        pltpu.sync_copy(
            zero_hbm.at[pl.ds(sid * rsub, rsub)],
            acc_sh.at[pl.ds(sid * rsub, rsub)],
        )
        plsc.subcore_barrier()
        pltpu.sync_copy(idx_hbm.at[cid, sid], idx_v)
        base = sid * (cpw * ck)

        def adds(g, rows_v):
            for b in range(_GRP):
                pltpu.sync_copy(
                    rows_v.at[pl.ds(b * ck, ck)],
                    acc_sh.at[idx_v.at[g + b]],
                    add=True,
                )

        pltpu.async_copy(vals_hbm.at[pl.ds(base, gsz)], rows_a, sem_a)

        def pair(pi, carry):
            ga = pi * (2 * _GRP)
            gb = ga + _GRP
            pltpu.make_async_copy(
                vals_hbm.at[pl.ds(base, gsz)], rows_a, sem_a).wait()
            pltpu.async_copy(
                vals_hbm.at[pl.ds(base + gb * ck, gsz)], rows_b, sem_b)
            adds(ga, rows_a)
            pltpu.make_async_copy(
                vals_hbm.at[pl.ds(base, gsz)], rows_b, sem_b).wait()

            @pl.when(pi < npair - 1)
            def _():
                pltpu.async_copy(
                    vals_hbm.at[pl.ds(base + (gb + _GRP) * ck, gsz)],
                    rows_a, sem_a)

            adds(gb, rows_b)
            return carry

        lax.fori_loop(0, npair, pair, 0)
        plsc.subcore_barrier()
        pltpu.sync_copy(
            acc_sh.at[pl.ds(sid * rsub, rsub)],
            out_hbm.at[cid, pl.ds(sid * rsub, rsub)],
        )

    return pl.kernel(
        body,
        out_type=jax.ShapeDtypeStruct((nc, rl, d), jnp.float32),
        mesh=mesh,
        scratch_types=[
            pltpu.VMEM((cpw, ck), jnp.int32),
            pltpu.VMEM((gsz, d), jnp.float32),
            pltpu.VMEM((gsz, d), jnp.float32),
            pltpu.VMEM_SHARED((rl, d), jnp.float32),
            pltpu.SemaphoreType.DMA,
            pltpu.SemaphoreType.DMA,
        ],
    )(vals, idx4, zeros_init)


def _ln(x, g, b):
    m = jnp.mean(x, axis=-1, keepdims=True)
    v = jnp.mean((x - m) ** 2, axis=-1, keepdims=True)
    return (x - m) * lax.rsqrt(v + 1e-5) * g + b


def _dot(a, b):
    return jnp.dot(a, b, preferred_element_type=jnp.float32,
                   precision=lax.Precision.DEFAULT)


def _full(shape):
    return pl.BlockSpec(shape, lambda i: (0,) * len(shape))


def _mlp_tail(h1, w2, b2, g2, be2, w3, b3):
    h2 = jnp.maximum(_ln(_dot(h1, w2) + b2, g2, be2), 0.0)
    return _dot(h2, w3) + b3, h2


def _pad64(x, blk):
    return jnp.concatenate([x, jnp.zeros((blk, 64), jnp.float32)], axis=1)


def _tc_input_mlp(x, p, blk):
    r = x.shape[0]
    (w1, b1, g1, be1), (w2, b2, g2, be2) = p["hid"]
    w3, b3 = p["out"]

    def body(x_ref, w1_r, b1_r, g1_r, be1_r, w2_r, b2_r, g2_r, be2_r,
             w3_r, b3_r, out_ref):
        h1 = jnp.maximum(_ln(_dot(x_ref[...], w1_r[...]) + b1_r[...],
                             g1_r[...], be1_r[...]), 0.0)
        out, _ = _mlp_tail(h1, w2_r[...], b2_r[...], g2_r[...], be2_r[...],
                           w3_r[...], b3_r[...])
        out_ref[...] = _pad64(out, blk)

    return pl.pallas_call(
        body,
        grid=(r // blk,),
        in_specs=[
            pl.BlockSpec((blk, 128), lambda i: (i, 0)),
            _full(w1.shape), _full(b1.shape), _full(g1.shape), _full(be1.shape),
            _full(w2.shape), _full(b2.shape), _full(g2.shape), _full(be2.shape),
            _full(w3.shape), _full(b3.shape),
        ],
        out_specs=pl.BlockSpec((blk, 128), lambda i: (i, 0)),
        out_shape=jax.ShapeDtypeStruct((r, 128), jnp.float32),
    )(x, w1, b1, g1, be1, w2, b2, g2, be2, w3, b3)


def _tc_msg_mlp(sg, ce, p, blk):
    e = sg.shape[0]
    w1s, w1c, b1, g1, be1 = p["l1"]
    (w2, b2, g2, be2) = p["l2"]
    w3, b3 = p["out"]

    def body(sg_ref, ce_ref, w1s_r, w1c_r, b1_r, g1_r, be1_r,
             w2_r, b2_r, g2_r, be2_r, w3_r, b3_r, out_ref):
        pre1 = _dot(sg_ref[...], w1s_r[...]) + _dot(ce_ref[...], w1c_r[...]) + b1_r[...]
        h1 = jnp.maximum(_ln(pre1, g1_r[...], be1_r[...]), 0.0)
        out, _ = _mlp_tail(h1, w2_r[...], b2_r[...], g2_r[...], be2_r[...],
                           w3_r[...], b3_r[...])
        out_ref[...] = jnp.concatenate(
            [jnp.tanh(out),
             jnp.ones((blk, 1), jnp.float32),
             jnp.zeros((blk, 63), jnp.float32)], axis=1)

    return pl.pallas_call(
        body,
        grid=(e // blk,),
        in_specs=[
            pl.BlockSpec((blk, 128), lambda i: (i, 0)),
            pl.BlockSpec((blk, 128), lambda i: (i, 0)),
            _full(w1s.shape), _full(w1c.shape), _full(b1.shape),
            _full(g1.shape), _full(be1.shape),
            _full(w2.shape), _full(b2.shape), _full(g2.shape), _full(be2.shape),
            _full(w3.shape), _full(b3.shape),
        ],
        out_specs=pl.BlockSpec((blk, 128), lambda i: (i, 0)),
        out_shape=jax.ShapeDtypeStruct((e, 128), jnp.float32),
    )(sg, ce, w1s, w1c, b1, g1, be1, w2, b2, g2, be2, w3, b3)


def _tc_update_mlp(msum, st, p, blk):
    r = st.shape[0]
    nbh = (r // 2) // blk
    w1m, w1s, b1, g1, be1 = p["l1"]
    (w2, b2, g2, be2) = p["l2"]
    w3, b3 = p["out"]

    def body(ms_ref, st_ref, w1m_r, w1s_r, b1_r, g1_r, be1_r,
             w2_r, b2_r, g2_r, be2_r, w3_r, b3_r, out_ref):
        m = ms_ref[0]
        inv = 1.0 / jnp.maximum(m[:, 64:65], 1.0)
        pre1 = (_dot(m * inv, w1m_r[...])
                + _dot(st_ref[...], w1s_r[...]) + b1_r[...])
        h1 = jnp.maximum(_ln(pre1, g1_r[...], be1_r[...]), 0.0)
        out, _ = _mlp_tail(h1, w2_r[...], b2_r[...], g2_r[...], be2_r[...],
                           w3_r[...], b3_r[...])
        out_ref[...] = _pad64(out, blk)

    return pl.pallas_call(
        body,
        grid=(r // blk,),
        in_specs=[
            pl.BlockSpec((1, blk, 128), lambda i: (i // nbh, i % nbh, 0)),
            pl.BlockSpec((blk, 128), lambda i: (i, 0)),
            _full(w1m.shape), _full(w1s.shape), _full(b1.shape),
            _full(g1.shape), _full(be1.shape),
            _full(w2.shape), _full(b2.shape), _full(g2.shape), _full(be2.shape),
            _full(w3.shape), _full(b3.shape),
        ],
        out_specs=pl.BlockSpec((blk, 128), lambda i: (i, 0)),
        out_shape=jax.ShapeDtypeStruct((r, 128), jnp.float32),
    )(msum, st, w1m, w1s, b1, g1, be1, w2, b2, g2, be2, w3, b3)


def _tc_output_mlp(st, p, blk, n_valid):
    r = st.shape[0]
    (w1, b1, g1, be1), (w2, b2, g2, be2) = p["hid"]
    w3, b3 = p["out"]
    wsig, bsig = p["sig"]

    def body(st_ref, w1_r, b1_r, g1_r, be1_r, w2_r, b2_r, g2_r, be2_r,
             w3_r, b3_r, ws_r, bs_r, act_ref, psum_ref):
        i = pl.program_id(0)
        h1 = jnp.maximum(_ln(_dot(st_ref[...], w1_r[...]) + b1_r[...],
                             g1_r[...], be1_r[...]), 0.0)
        out, h2 = _mlp_tail(h1, w2_r[...], b2_r[...], g2_r[...], be2_r[...],
                            w3_r[...], b3_r[...])
        act_ref[...] = jnp.tanh(out)
        sig = jax.nn.sigmoid(_dot(h2, ws_r[...]) + bs_r[...])
        rows = i * blk + lax.broadcasted_iota(jnp.int32, (blk, 1), 0)
        sig = jnp.where(rows < n_valid, sig, 0.0)
        psum_ref[...] = jnp.sum(sig).reshape(1, 1, 1)

    return pl.pallas_call(
        body,
        grid=(r // blk,),
        in_specs=[
            pl.BlockSpec((blk, 128), lambda i: (i, 0)),
            _full(w1.shape), _full(b1.shape), _full(g1.shape), _full(be1.shape),
            _full(w2.shape), _full(b2.shape), _full(g2.shape), _full(be2.shape),
            _full(w3.shape), _full(b3.shape),
            _full(wsig.shape), _full(bsig.shape),
        ],
        out_specs=[
            pl.BlockSpec((blk, 1), lambda i: (i, 0)),
            pl.BlockSpec((1, 1, 1), lambda i: (i, 0, 0)),
        ],
        out_shape=[
            jax.ShapeDtypeStruct((r, 1), jnp.float32),
            jax.ShapeDtypeStruct((r // blk, 1, 1), jnp.float32),
        ],
    )(st, w1, b1, g1, be1, w2, b2, g2, be2, w3, b3, wsig, bsig)


def _row2(v):
    return v.reshape(1, -1)


def _padrows(w, rows):
    return jnp.zeros((rows, w.shape[1]), jnp.float32).at[:w.shape[0]].set(w)


def kernel(state, node_features, edge_feature, edge_index, params_input,
           params_message, params_update, params_output):
    f32 = jnp.float32
    b, sd = state.shape
    nsv = sd // 2
    n = (nsv - 5) // 2
    e = edge_index.shape[1]

    blk_n = 1024
    n_pad = -(-(n + 1) // (2 * blk_n)) * (2 * blk_n)
    n_half = n_pad // 2
    rl = n_half + _DUMP
    epw = _GRP * _CK
    e_pad = -(-e // (_NW * epw)) * (_NW * epw)
    cpw = e_pad // (_NW * _CK)
    cpw2 = e_pad // (_NS * _CK)
    blk_e = 1024

    s0 = state[0]
    glob = jnp.concatenate([s0[0:5], s0[nsv:nsv + 5]])
    node_input = jnp.concatenate([
        node_features,
        jnp.broadcast_to(glob[None, :], (n, 10)),
        s0[5:5 + n][:, None],
        s0[5 + n:5 + 2 * n][:, None],
        s0[nsv + 5:nsv + 5 + n][:, None],
        s0[nsv + 5 + n:nsv + 5 + 2 * n][:, None],
    ], axis=1)
    ninp_tbl = jnp.zeros((n_pad, 128), f32).at[:n, :20].set(node_input)

    src = jnp.zeros((e_pad,), jnp.int32).at[:e].set(edge_index[0])
    dst = jnp.full((e_pad,), n, jnp.int32).at[:e].set(edge_index[1])
    src3 = src.reshape(_NW, cpw, _CK)
    dst2 = dst.reshape(_NS, cpw2, _CK)
    halves = []
    for c in range(_NC):
        loc = dst2 - c * n_half
        ok = (loc >= 0) & (loc < n_half)
        halves.append(jnp.where(ok, loc, n_half))
    dst4 = jnp.stack(halves)
    edat = jnp.zeros((e_pad,), f32).at[:e].set(edge_feature)

    def prep_plain(p, in_rows):
        (w1, b1, g1, be1), (w2, b2, g2, be2) = p["hidden"]
        return {
            "hid": [
                (_padrows(w1, in_rows), _row2(b1), _row2(g1), _row2(be1)),
                (w2, _row2(b2), _row2(g2), _row2(be2)),
            ],
            "out": (p["Wout"], _row2(p["bout"])),
        }

    pi = prep_plain(params_input, 128)

    wm1, bm1, gm1, bem1 = params_message["hidden"][0]
    pm = {
        "l1": (_padrows(wm1[:64], 128),
               _padrows(wm1[65:85], 128).at[20].set(wm1[64]),
               _row2(bm1), _row2(gm1), _row2(bem1)),
        "l2": tuple([params_message["hidden"][1][0]]
                    + [_row2(v) for v in params_message["hidden"][1][1:]]),
        "out": (params_message["Wout"], _row2(params_message["bout"])),
    }
    wu1, bu1, gu1, beu1 = params_update["hidden"][0]
    pu = {
        "l1": (_padrows(wu1[:64], 128), _padrows(wu1[64:], 128),
               _row2(bu1), _row2(gu1), _row2(beu1)),
        "l2": tuple([params_update["hidden"][1][0]]
                    + [_row2(v) for v in params_update["hidden"][1][1:]]),
        "out": (params_update["Wout"], _row2(params_update["bout"])),
    }
    po = prep_plain(params_output, 128)
    po["sig"] = (params_output["Wsig"], _row2(params_output["bsig"]))

    ce = _sc_gather(ninp_tbl, src3)
    ce = ce.at[:, 20].set(edat)

    node_state = _tc_input_mlp(ninp_tbl, pi, blk_n)

    zrl = jnp.zeros((rl, 128), f32)
    for _ in range(6):
        sg = _sc_gather(node_state, src3)
        msg = _tc_msg_mlp(sg, ce, pm, blk_e)
        msum = _sc_scatter_add(msg, dst4, zrl)
        node_state = _tc_update_mlp(msum, node_state, pu, blk_n)

    act, psum = _tc_output_mlp(node_state, po, blk_n, n)
    actions = act[:n, 0][None, :]
    sigmoids = (jnp.sum(psum) / n).reshape(1)
    return (actions, sigmoids)
